# baseline (speedup 1.0000x reference)
import functools

import jax
import jax.numpy as jnp
from jax import lax
from jax.experimental import pallas as pl
from jax.experimental.pallas import tpu as pltpu
from jax.experimental.pallas import tpu_sc as plsc

NSTU = 8000
NEXER = 1500
NKNOW = 500
NNODE = 10000
D = 128
NEDGE = 160000
BATCH = 4096
NLAYER = 3

NCORE = 2
NSUB = 16
LANES = 16

CH = 128
CPT = 80
GRP = 40
EPAD = NSUB * CPT * CH
RPT = 624
BPT = BATCH // NSUB
BCH = BPT // CH
NVREG = D // LANES


def _scale_rows(buf, vbuf, vrow):
    def body(i, _):
        v16 = vbuf[vrow, pl.ds(i * LANES, LANES)]
        for l in range(LANES):
            e = i * LANES + l
            v = v16[l]
            for d in range(NVREG):
                sl = pl.ds(d * LANES, LANES)
                buf[e, sl] = buf[e, sl] * v
        return 0
    lax.fori_loop(0, CH // LANES, body, 0, unroll=False)


def _sum2(g0, g1):
    def body(e, _):
        for d in range(NVREG):
            sl = pl.ds(d * LANES, LANES)
            g0[e, sl] = g0[e, sl] + g1[e, sl]
        return 0
    lax.fori_loop(0, CH, body, 0, unroll=False)


_ZCH = ((0, CH), (CH, CH), (2 * CH, CH), (3 * CH, CH), (4 * CH, RPT - 4 * CH))
_TAIL = NNODE - RPT * NSUB


def _spmm_chain(sid, x_hbm, epk_h, vals_h, olay, acc, ebuf, vbuf,
                bufa, bufb, sia, sib, gsem, ssem):
    def zbody(e, _):
        for d in range(NVREG):
            bufa[e, pl.ds(d * LANES, LANES)] = jnp.zeros((LANES,), jnp.float32)
        return 0

    def layer(k, _):
        lax.fori_loop(0, CH, zbody, 0, unroll=False)
        for off, sz in _ZCH:
            pltpu.sync_copy(bufa.at[pl.ds(0, sz)],
                            acc.at[pl.ds(sid * RPT + off, sz)])

        @pl.when(sid == NSUB - 1)
        def _():
            pltpu.sync_copy(bufa.at[pl.ds(0, _TAIL)],
                            acc.at[pl.ds(RPT * NSUB, _TAIL)])
        plsc.subcore_barrier()

        def gload(c):
            off = pl.multiple_of(c * 2, 2 * GRP)
            pltpu.sync_copy(epk_h.at[sid].at[pl.ds(off, 2 * GRP)], ebuf)
            voff = pl.multiple_of(c, GRP)
            pltpu.sync_copy(vals_h.at[sid].at[pl.ds(voff, GRP)], vbuf)

        def cpidx(rowi, si):
            for d in range(NVREG):
                sl = pl.ds(d * LANES, LANES)
                si[0, sl] = ebuf[rowi, sl]

        def gat(islot, buf):
            @pl.when(k == 0)
            def _():
                pltpu.async_copy(x_hbm.at[ebuf.at[islot]], buf, gsem)

            @pl.when(k != 0)
            def _():
                pltpu.async_copy(olay.at[k - 1].at[ebuf.at[islot]], buf, gsem)

        def gwait(buf):
            pltpu.make_async_copy(x_hbm.at[ebuf.at[1]], buf, gsem).wait()

        def swait(buf, si):
            pltpu.make_async_copy(buf, acc.at[si.at[0]], ssem).wait()

        def pair(t, _):
            c0 = 2 * t
            jj0 = lax.rem(c0, GRP)

            @pl.when(jj0 == 0)
            def _():
                gload(c0)
                gat(1, bufa)

            gwait(bufa)

            @pl.when(t > 0)
            def _():
                swait(bufb, sib)

            gat(2 * jj0 + 3, bufb)
            cpidx(2 * jj0, sia)
            _scale_rows(bufa, vbuf, jj0)
            pltpu.async_copy(bufa, acc.at[sia.at[0]], ssem, add=True)
            gwait(bufb)
            cpidx(2 * jj0 + 2, sib)
            _scale_rows(bufb, vbuf, jj0 + 1)
            swait(bufa, sia)

            @pl.when(jj0 != GRP - 2)
            def _():
                gat(2 * jj0 + 5, bufa)
            pltpu.async_copy(bufb, acc.at[sib.at[0]], ssem, add=True)
            return 0
        lax.fori_loop(0, CPT // 2, pair, 0, unroll=False)
        swait(bufb, sib)
        plsc.subcore_barrier()

        pltpu.sync_copy(acc.at[pl.ds(sid * RPT, RPT)],
                        olay.at[k].at[pl.ds(sid * RPT, RPT)])

        @pl.when(sid == NSUB - 1)
        def _():
            pltpu.sync_copy(acc.at[pl.ds(RPT * NSUB, _TAIL)],
                            olay.at[k].at[pl.ds(RPT * NSUB, _TAIL)])
        plsc.subcore_barrier()
        return 0
    lax.fori_loop(0, NLAYER, layer, 0, unroll=False)


def _batch_gather(sid, idx_h, offset, x_hbm, o_hbm, gx_hbm, gsum_hbm,
                  bidx, g0, g1, sem):
    base = sid * BPT
    for j in range(BCH):
        pltpu.sync_copy(idx_h.at[pl.ds(base + j * CH, CH)], bidx.at[j])
        if offset:
            def obody(i, _):
                sl = pl.ds(i * LANES, LANES)
                bidx[j, sl] = bidx[j, sl] + offset
                return 0
            lax.fori_loop(0, CH // LANES, obody, 0, unroll=False)
        idx = bidx.at[j]
        if gx_hbm is not None:
            pltpu.async_copy(x_hbm.at[idx], g0, sem)
            pltpu.async_copy(o_hbm.at[0].at[idx], g1, sem)
            pltpu.make_async_copy(x_hbm.at[idx], g0, sem).wait()
            pltpu.sync_copy(g0, gx_hbm.at[pl.ds(base + j * CH, CH)])
            pltpu.make_async_copy(o_hbm.at[0].at[idx], g1, sem).wait()
        else:
            pltpu.async_copy(o_hbm.at[0].at[idx], g1, sem)
            pltpu.async_copy(o_hbm.at[1].at[idx], g0, sem)
            pltpu.make_async_copy(o_hbm.at[0].at[idx], g1, sem).wait()
        if gx_hbm is not None:
            pltpu.async_copy(o_hbm.at[1].at[idx], g0, sem)
            pltpu.make_async_copy(o_hbm.at[1].at[idx], g0, sem).wait()
        else:
            pltpu.make_async_copy(o_hbm.at[1].at[idx], g0, sem).wait()
        _sum2(g1, g0)
        pltpu.async_copy(o_hbm.at[2].at[idx], g0, sem)
        pltpu.make_async_copy(o_hbm.at[2].at[idx], g0, sem).wait()
        _sum2(g1, g0)
        pltpu.sync_copy(g1, gsum_hbm.at[pl.ds(base + j * CH, CH)])


def _sc_body(x_hbm, repk, rvals, wepk, wvals, sid_h, eid_h,
             o_r, o_w, gsx, gsr, gsw, gex, ger, gew,
             acc, ebuf, vbuf, bufa, bufb, sia, sib, bidx, gsem, ssem):
    cid = lax.axis_index("c")
    sid = lax.axis_index("s")

    @pl.when(cid == 0)
    def _():
        _spmm_chain(sid, x_hbm, repk, rvals, o_r, acc, ebuf, vbuf,
                    bufa, bufb, sia, sib, gsem, ssem)
        _batch_gather(sid, sid_h, 0, x_hbm, o_r, gsx, gsr,
                      bidx, bufa, bufb, gsem)
        _batch_gather(sid, eid_h, NSTU, x_hbm, o_r, None, ger,
                      bidx, bufa, bufb, gsem)

    @pl.when(cid == 1)
    def _():
        _spmm_chain(sid, x_hbm, wepk, wvals, o_w, acc, ebuf, vbuf,
                    bufa, bufb, sia, sib, gsem, ssem)
        _batch_gather(sid, sid_h, 0, x_hbm, o_w, None, gsw,
                      bidx, bufa, bufb, gsem)
        _batch_gather(sid, eid_h, NSTU, x_hbm, o_w, gex, gew,
                      bidx, bufa, bufb, gsem)


_sc_call = functools.partial(
    pl.kernel,
    out_type=(
        jax.ShapeDtypeStruct((NLAYER, NNODE, D), jnp.float32),
        jax.ShapeDtypeStruct((NLAYER, NNODE, D), jnp.float32),
        jax.ShapeDtypeStruct((BATCH, D), jnp.float32),
        jax.ShapeDtypeStruct((BATCH, D), jnp.float32),
        jax.ShapeDtypeStruct((BATCH, D), jnp.float32),
        jax.ShapeDtypeStruct((BATCH, D), jnp.float32),
        jax.ShapeDtypeStruct((BATCH, D), jnp.float32),
        jax.ShapeDtypeStruct((BATCH, D), jnp.float32),
    ),
    mesh=plsc.VectorSubcoreMesh(core_axis_name="c", subcore_axis_name="s"),
    scratch_types=(
        pltpu.VMEM_SHARED((NNODE, D), jnp.float32),
        pltpu.VMEM((2 * GRP, CH), jnp.int32),
        pltpu.VMEM((GRP, CH), jnp.float32),
        pltpu.VMEM((CH, D), jnp.float32),
        pltpu.VMEM((CH, D), jnp.float32),
        pltpu.VMEM((1, CH), jnp.int32),
        pltpu.VMEM((1, CH), jnp.int32),
        pltpu.VMEM((BCH, CH), jnp.int32),
        pltpu.SemaphoreType.DMA,
        pltpu.SemaphoreType.DMA,
    ),
)(_sc_body)


def _tc_batch_body(gsx, gsr, gsw, gex, ger, gew, wrT, wwT, bc, wtsT, bts,
                   wteT, bte, eid2, discT, stu_o, diff_o, disc_o):
    bcv = bc[...] * 0.75
    iot = lax.broadcasted_iota(jnp.int32, (_BROWS, NEXER), 1)
    msk = iot == eid2[...]
    disc_o[...] = jnp.sum(jnp.where(msk, discT[...], 0.0), axis=1,
                          keepdims=True)
    sfeat = (gsx[...] * 0.25
             + (jnp.dot(gsr[...], wrT[...], preferred_element_type=jnp.float32)
                + jnp.dot(gsw[...], wwT[...], preferred_element_type=jnp.float32)) * 0.25
             + bcv)
    stu_o[...] = jnp.dot(sfeat, wtsT[...], preferred_element_type=jnp.float32) + bts[...]
    efeat = (gex[...] * 0.25
             + (jnp.dot(ger[...], wrT[...], preferred_element_type=jnp.float32)
                + jnp.dot(gew[...], wwT[...], preferred_element_type=jnp.float32)) * 0.25
             + bcv)
    diff_o[...] = jnp.dot(efeat, wteT[...], preferred_element_type=jnp.float32) + bte[...]


def _tc_know_body(o_r, o_w, xk, wrT, wwT, bc, wtkT, btk, know_o):
    rsum = o_r[0, NKNOW:] + o_r[1, NKNOW:] + o_r[2, NKNOW:]
    wsum = o_w[0, NKNOW:] + o_w[1, NKNOW:] + o_w[2, NKNOW:]
    feat = (xk[...] * 0.25
            + (jnp.dot(rsum, wrT[...], preferred_element_type=jnp.float32)
               + jnp.dot(wsum, wwT[...], preferred_element_type=jnp.float32)) * 0.25
            + bc[...] * 0.75)
    know_o[...] = jnp.dot(feat, wtkT[...], preferred_element_type=jnp.float32) + btk[...]


_BROWS = 512
_NBLK = BATCH // _BROWS


def _pack_edges(idx, vals):
    r = idx[0].astype(jnp.int32)
    c = idx[1].astype(jnp.int32)
    v = vals.astype(jnp.float32)
    pad = EPAD - NEDGE
    r = jnp.pad(r, (0, pad)).reshape(NSUB, CPT, 1, CH)
    c = jnp.pad(c, (0, pad)).reshape(NSUB, CPT, 1, CH)
    v = jnp.pad(v, (0, pad)).reshape(NSUB, CPT, CH)
    return jnp.concatenate([r, c], axis=2).reshape(NSUB, 2 * CPT, CH), v


def kernel(right_index, right_values, wrong_index, wrong_values, student_id,
           exercise_id, stu_emb, exer_emb, know_emb, disc_emb, W_concat,
           b_concat, W_ts, b_ts, W_te, b_te, W_tk, b_tk):
    x = jnp.concatenate([stu_emb, exer_emb, know_emb], axis=0)
    repk, rvals = _pack_edges(right_index, right_values)
    wepk, wvals = _pack_edges(wrong_index, wrong_values)
    sid = student_id.astype(jnp.int32)
    eid = exercise_id.astype(jnp.int32)
    disc = disc_emb.astype(jnp.float32)

    (o_r, o_w, gsx, gsr, gsw, gex, ger, gew) = _sc_call(
        x, repk, rvals, wepk, wvals, sid, eid)

    wrT = W_concat[:, :D].T
    wwT = W_concat[:, D:].T
    bc = b_concat.reshape(1, D)
    wtsT = W_ts.T
    wteT = W_te.T
    wtkT = W_tk.T
    bts = b_ts.reshape(1, NKNOW)
    bte = b_te.reshape(1, NKNOW)
    btk = b_tk.reshape(1, NKNOW)

    full = pl.BlockSpec((None,) * 0 + (D, D), lambda i: (0, 0))
    wts_spec = pl.BlockSpec((D, NKNOW), lambda i: (0, 0))
    bvec = pl.BlockSpec((1, NKNOW), lambda i: (0, 0))
    gspec = pl.BlockSpec((_BROWS, D), lambda i: (i, 0))

    student_ts, diff_ts, disc_ts = pl.pallas_call(
        _tc_batch_body,
        grid=(_NBLK,),
        in_specs=[gspec, gspec, gspec, gspec, gspec, gspec,
                  full, full, pl.BlockSpec((1, D), lambda i: (0, 0)),
                  wts_spec, bvec, wts_spec, bvec,
                  pl.BlockSpec((_BROWS, 1), lambda i: (i, 0)),
                  pl.BlockSpec((1, NEXER), lambda i: (0, 0))],
        out_specs=[pl.BlockSpec((_BROWS, NKNOW), lambda i: (i, 0)),
                   pl.BlockSpec((_BROWS, NKNOW), lambda i: (i, 0)),
                   pl.BlockSpec((_BROWS, 1), lambda i: (i, 0))],
        out_shape=[jax.ShapeDtypeStruct((BATCH, NKNOW), jnp.float32),
                   jax.ShapeDtypeStruct((BATCH, NKNOW), jnp.float32),
                   jax.ShapeDtypeStruct((BATCH, 1), jnp.float32)],
    )(gsx, gsr, gsw, gex, ger, gew, wrT, wwT, bc, wtsT, bts, wteT, bte,
      eid[:, None], disc.reshape(1, NEXER))

    xk = x[NSTU + NEXER:]
    ospec = pl.BlockSpec((NLAYER, 2 * NKNOW, D),
                         lambda i: (0, NNODE // (2 * NKNOW) - 1, 0))
    knowledge_ts = pl.pallas_call(
        _tc_know_body,
        grid=(1,),
        in_specs=[ospec, ospec,
                  pl.BlockSpec((NKNOW, D), lambda i: (0, 0)),
                  pl.BlockSpec((D, D), lambda i: (0, 0)),
                  pl.BlockSpec((D, D), lambda i: (0, 0)),
                  pl.BlockSpec((1, D), lambda i: (0, 0)),
                  pl.BlockSpec((D, NKNOW), lambda i: (0, 0)),
                  pl.BlockSpec((1, NKNOW), lambda i: (0, 0))],
        out_specs=pl.BlockSpec((NKNOW, NKNOW), lambda i: (0, 0)),
        out_shape=jax.ShapeDtypeStruct((NKNOW, NKNOW), jnp.float32),
    )(o_r, o_w, xk, wrT, wwT, bc, wtkT, btk)

    return (student_ts, diff_ts, disc_ts, knowledge_ts)


# speedup vs baseline: 1.0442x; 1.0442x over previous
import functools

import jax
import jax.numpy as jnp
from jax import lax
from jax.experimental import pallas as pl
from jax.experimental.pallas import tpu as pltpu
from jax.experimental.pallas import tpu_sc as plsc

NSTU = 8000
NEXER = 1500
NKNOW = 500
NNODE = 10000
D = 128
NEDGE = 160000
BATCH = 4096
NLAYER = 3

NCORE = 2
NSUB = 16
LANES = 16

CH = 128
CPT = 80
GRP = 40
EPAD = NSUB * CPT * CH
RPT = 624
BPT = BATCH // NSUB
BCH = BPT // CH
NVREG = D // LANES


def _scale_rows(buf, vbuf, vrow):
    def body(i, _):
        v16 = vbuf[vrow, pl.ds(i * LANES, LANES)]
        for l in range(LANES):
            e = i * LANES + l
            v = v16[l]
            for d in range(NVREG):
                sl = pl.ds(d * LANES, LANES)
                buf[e, sl] = buf[e, sl] * v
        return 0
    lax.fori_loop(0, CH // LANES, body, 0, unroll=False)


def _sum2(g0, g1):
    def body(e, _):
        for d in range(NVREG):
            sl = pl.ds(d * LANES, LANES)
            g0[e, sl] = g0[e, sl] + g1[e, sl]
        return 0
    lax.fori_loop(0, CH, body, 0, unroll=False)


_ZCH = ((0, CH), (CH, CH), (2 * CH, CH), (3 * CH, CH), (4 * CH, RPT - 4 * CH))
_TAIL = NNODE - RPT * NSUB


def _spmm_chain(sid, x_hbm, epk_h, vals_h, olay, acc, ebuf, vbuf,
                bufa, bufb, sia, sib, gsem, ssem):
    for off, sz in _ZCH:
        pltpu.sync_copy(x_hbm.at[pl.ds(sid * RPT + off, sz)],
                        bufa.at[pl.ds(0, sz)])
        pltpu.sync_copy(bufa.at[pl.ds(0, sz)],
                        olay.at[0].at[pl.ds(sid * RPT + off, sz)])

    @pl.when(sid == NSUB - 1)
    def _():
        pltpu.sync_copy(x_hbm.at[pl.ds(RPT * NSUB, _TAIL)],
                        bufa.at[pl.ds(0, _TAIL)])
        pltpu.sync_copy(bufa.at[pl.ds(0, _TAIL)],
                        olay.at[0].at[pl.ds(RPT * NSUB, _TAIL)])

    def zbody(e, _):
        for d in range(NVREG):
            bufa[e, pl.ds(d * LANES, LANES)] = jnp.zeros((LANES,), jnp.float32)
        return 0

    def layer(k, _):
        src = olay.at[k]
        lax.fori_loop(0, CH, zbody, 0, unroll=False)
        for off, sz in _ZCH:
            pltpu.sync_copy(bufa.at[pl.ds(0, sz)],
                            acc.at[pl.ds(sid * RPT + off, sz)])

        @pl.when(sid == NSUB - 1)
        def _():
            pltpu.sync_copy(bufa.at[pl.ds(0, _TAIL)],
                            acc.at[pl.ds(RPT * NSUB, _TAIL)])
        plsc.subcore_barrier()

        def gload(c):
            off = pl.multiple_of(c * 2, 2 * GRP)
            pltpu.sync_copy(epk_h.at[sid].at[pl.ds(off, 2 * GRP)], ebuf)
            voff = pl.multiple_of(c, GRP)
            pltpu.sync_copy(vals_h.at[sid].at[pl.ds(voff, GRP)], vbuf)

        def cpidx(rowi, si):
            for d in range(NVREG):
                sl = pl.ds(d * LANES, LANES)
                si[0, sl] = ebuf[rowi, sl]

        def gwait(buf):
            pltpu.make_async_copy(src.at[ebuf.at[1]], buf, gsem).wait()

        def swait(buf, si):
            pltpu.make_async_copy(buf, acc.at[si.at[0]], ssem).wait()

        def pair(t, _):
            c0 = 2 * t
            jj0 = lax.rem(c0, GRP)

            @pl.when(jj0 == 0)
            def _():
                gload(c0)
                pltpu.async_copy(src.at[ebuf.at[1]], bufa, gsem)

            gwait(bufa)

            @pl.when(t > 0)
            def _():
                swait(bufb, sib)

            pltpu.async_copy(src.at[ebuf.at[2 * jj0 + 3]], bufb, gsem)
            cpidx(2 * jj0, sia)
            _scale_rows(bufa, vbuf, jj0)
            pltpu.async_copy(bufa, acc.at[sia.at[0]], ssem, add=True)
            gwait(bufb)
            cpidx(2 * jj0 + 2, sib)
            _scale_rows(bufb, vbuf, jj0 + 1)
            swait(bufa, sia)

            @pl.when(jj0 != GRP - 2)
            def _():
                pltpu.async_copy(src.at[ebuf.at[2 * jj0 + 5]], bufa, gsem)
            pltpu.async_copy(bufb, acc.at[sib.at[0]], ssem, add=True)
            return 0
        lax.fori_loop(0, CPT // 2, pair, 0, unroll=False)
        swait(bufb, sib)
        plsc.subcore_barrier()

        pltpu.sync_copy(acc.at[pl.ds(sid * RPT, RPT)],
                        olay.at[k + 1].at[pl.ds(sid * RPT, RPT)])

        @pl.when(sid == NSUB - 1)
        def _():
            pltpu.sync_copy(acc.at[pl.ds(RPT * NSUB, _TAIL)],
                            olay.at[k + 1].at[pl.ds(RPT * NSUB, _TAIL)])
        plsc.subcore_barrier()
        return 0
    lax.fori_loop(0, NLAYER, layer, 0, unroll=False)


def _batch_gather(sid, idx_h, offset, x_hbm, o_hbm, gx_hbm, gsum_hbm,
                  bidx, g0, g1, sem):
    base = sid * BPT
    for j in range(BCH):
        pltpu.sync_copy(idx_h.at[pl.ds(base + j * CH, CH)], bidx.at[j])
        if offset:
            def obody(i, _):
                sl = pl.ds(i * LANES, LANES)
                bidx[j, sl] = bidx[j, sl] + offset
                return 0
            lax.fori_loop(0, CH // LANES, obody, 0, unroll=False)
        idx = bidx.at[j]
        if gx_hbm is not None:
            pltpu.async_copy(x_hbm.at[idx], g0, sem)
            pltpu.async_copy(o_hbm.at[1].at[idx], g1, sem)
            pltpu.make_async_copy(x_hbm.at[idx], g0, sem).wait()
            pltpu.sync_copy(g0, gx_hbm.at[pl.ds(base + j * CH, CH)])
            pltpu.make_async_copy(o_hbm.at[1].at[idx], g1, sem).wait()
            pltpu.async_copy(o_hbm.at[2].at[idx], g0, sem)
            pltpu.make_async_copy(o_hbm.at[2].at[idx], g0, sem).wait()
        else:
            pltpu.async_copy(o_hbm.at[1].at[idx], g1, sem)
            pltpu.async_copy(o_hbm.at[2].at[idx], g0, sem)
            pltpu.make_async_copy(o_hbm.at[1].at[idx], g1, sem).wait()
            pltpu.make_async_copy(o_hbm.at[2].at[idx], g0, sem).wait()
        _sum2(g1, g0)
        pltpu.async_copy(o_hbm.at[3].at[idx], g0, sem)
        pltpu.make_async_copy(o_hbm.at[3].at[idx], g0, sem).wait()
        _sum2(g1, g0)
        pltpu.sync_copy(g1, gsum_hbm.at[pl.ds(base + j * CH, CH)])


def _sc_body(x_hbm, repk, rvals, wepk, wvals, sid_h, eid_h,
             o_r, o_w, gsx, gsr, gsw, gex, ger, gew,
             acc, ebuf, vbuf, bufa, bufb, sia, sib, bidx, gsem, ssem):
    cid = lax.axis_index("c")
    sid = lax.axis_index("s")

    @pl.when(cid == 0)
    def _():
        _spmm_chain(sid, x_hbm, repk, rvals, o_r, acc, ebuf, vbuf,
                    bufa, bufb, sia, sib, gsem, ssem)
        _batch_gather(sid, sid_h, 0, x_hbm, o_r, gsx, gsr,
                      bidx, bufa, bufb, gsem)
        _batch_gather(sid, eid_h, NSTU, x_hbm, o_r, None, ger,
                      bidx, bufa, bufb, gsem)

    @pl.when(cid == 1)
    def _():
        _spmm_chain(sid, x_hbm, wepk, wvals, o_w, acc, ebuf, vbuf,
                    bufa, bufb, sia, sib, gsem, ssem)
        _batch_gather(sid, sid_h, 0, x_hbm, o_w, None, gsw,
                      bidx, bufa, bufb, gsem)
        _batch_gather(sid, eid_h, NSTU, x_hbm, o_w, gex, gew,
                      bidx, bufa, bufb, gsem)


_sc_call = functools.partial(
    pl.kernel,
    out_type=(
        jax.ShapeDtypeStruct((NLAYER + 1, NNODE, D), jnp.float32),
        jax.ShapeDtypeStruct((NLAYER + 1, NNODE, D), jnp.float32),
        jax.ShapeDtypeStruct((BATCH, D), jnp.float32),
        jax.ShapeDtypeStruct((BATCH, D), jnp.float32),
        jax.ShapeDtypeStruct((BATCH, D), jnp.float32),
        jax.ShapeDtypeStruct((BATCH, D), jnp.float32),
        jax.ShapeDtypeStruct((BATCH, D), jnp.float32),
        jax.ShapeDtypeStruct((BATCH, D), jnp.float32),
    ),
    mesh=plsc.VectorSubcoreMesh(core_axis_name="c", subcore_axis_name="s"),
    scratch_types=(
        pltpu.VMEM_SHARED((NNODE, D), jnp.float32),
        pltpu.VMEM((2 * GRP, CH), jnp.int32),
        pltpu.VMEM((GRP, CH), jnp.float32),
        pltpu.VMEM((CH, D), jnp.float32),
        pltpu.VMEM((CH, D), jnp.float32),
        pltpu.VMEM((1, CH), jnp.int32),
        pltpu.VMEM((1, CH), jnp.int32),
        pltpu.VMEM((BCH, CH), jnp.int32),
        pltpu.SemaphoreType.DMA,
        pltpu.SemaphoreType.DMA,
    ),
)(_sc_body)


def _tc_batch_body(gsx, gsr, gsw, gex, ger, gew, wrT, wwT, bc, wtsT, bts,
                   wteT, bte, eid2, discT, stu_o, diff_o, disc_o):
    bcv = bc[...] * 0.75
    iot = lax.broadcasted_iota(jnp.int32, (_BROWS, NEXER), 1)
    msk = iot == eid2[...]
    disc_o[...] = jnp.sum(jnp.where(msk, discT[...], 0.0), axis=1,
                          keepdims=True)
    sfeat = (gsx[...] * 0.25
             + (jnp.dot(gsr[...], wrT[...], preferred_element_type=jnp.float32)
                + jnp.dot(gsw[...], wwT[...], preferred_element_type=jnp.float32)) * 0.25
             + bcv)
    stu_o[...] = jnp.dot(sfeat, wtsT[...], preferred_element_type=jnp.float32) + bts[...]
    efeat = (gex[...] * 0.25
             + (jnp.dot(ger[...], wrT[...], preferred_element_type=jnp.float32)
                + jnp.dot(gew[...], wwT[...], preferred_element_type=jnp.float32)) * 0.25
             + bcv)
    diff_o[...] = jnp.dot(efeat, wteT[...], preferred_element_type=jnp.float32) + bte[...]


def _tc_know_body(o_r, o_w, xk, wrT, wwT, bc, wtkT, btk, know_o):
    rsum = o_r[1, NKNOW:] + o_r[2, NKNOW:] + o_r[3, NKNOW:]
    wsum = o_w[1, NKNOW:] + o_w[2, NKNOW:] + o_w[3, NKNOW:]
    feat = (xk[...] * 0.25
            + (jnp.dot(rsum, wrT[...], preferred_element_type=jnp.float32)
               + jnp.dot(wsum, wwT[...], preferred_element_type=jnp.float32)) * 0.25
            + bc[...] * 0.75)
    know_o[...] = jnp.dot(feat, wtkT[...], preferred_element_type=jnp.float32) + btk[...]


_BROWS = 512
_NBLK = BATCH // _BROWS


def _pack_edges(idx, vals):
    r = idx[0].astype(jnp.int32)
    c = idx[1].astype(jnp.int32)
    v = vals.astype(jnp.float32)
    pad = EPAD - NEDGE
    r = jnp.pad(r, (0, pad)).reshape(NSUB, CPT, 1, CH)
    c = jnp.pad(c, (0, pad)).reshape(NSUB, CPT, 1, CH)
    v = jnp.pad(v, (0, pad)).reshape(NSUB, CPT, CH)
    return jnp.concatenate([r, c], axis=2).reshape(NSUB, 2 * CPT, CH), v


def kernel(right_index, right_values, wrong_index, wrong_values, student_id,
           exercise_id, stu_emb, exer_emb, know_emb, disc_emb, W_concat,
           b_concat, W_ts, b_ts, W_te, b_te, W_tk, b_tk):
    x = jnp.concatenate([stu_emb, exer_emb, know_emb], axis=0)
    repk, rvals = _pack_edges(right_index, right_values)
    wepk, wvals = _pack_edges(wrong_index, wrong_values)
    sid = student_id.astype(jnp.int32)
    eid = exercise_id.astype(jnp.int32)
    disc = disc_emb.astype(jnp.float32)

    (o_r, o_w, gsx, gsr, gsw, gex, ger, gew) = _sc_call(
        x, repk, rvals, wepk, wvals, sid, eid)

    wrT = W_concat[:, :D].T
    wwT = W_concat[:, D:].T
    bc = b_concat.reshape(1, D)
    wtsT = W_ts.T
    wteT = W_te.T
    wtkT = W_tk.T
    bts = b_ts.reshape(1, NKNOW)
    bte = b_te.reshape(1, NKNOW)
    btk = b_tk.reshape(1, NKNOW)

    full = pl.BlockSpec((None,) * 0 + (D, D), lambda i: (0, 0))
    wts_spec = pl.BlockSpec((D, NKNOW), lambda i: (0, 0))
    bvec = pl.BlockSpec((1, NKNOW), lambda i: (0, 0))
    gspec = pl.BlockSpec((_BROWS, D), lambda i: (i, 0))

    student_ts, diff_ts, disc_ts = pl.pallas_call(
        _tc_batch_body,
        grid=(_NBLK,),
        in_specs=[gspec, gspec, gspec, gspec, gspec, gspec,
                  full, full, pl.BlockSpec((1, D), lambda i: (0, 0)),
                  wts_spec, bvec, wts_spec, bvec,
                  pl.BlockSpec((_BROWS, 1), lambda i: (i, 0)),
                  pl.BlockSpec((1, NEXER), lambda i: (0, 0))],
        out_specs=[pl.BlockSpec((_BROWS, NKNOW), lambda i: (i, 0)),
                   pl.BlockSpec((_BROWS, NKNOW), lambda i: (i, 0)),
                   pl.BlockSpec((_BROWS, 1), lambda i: (i, 0))],
        out_shape=[jax.ShapeDtypeStruct((BATCH, NKNOW), jnp.float32),
                   jax.ShapeDtypeStruct((BATCH, NKNOW), jnp.float32),
                   jax.ShapeDtypeStruct((BATCH, 1), jnp.float32)],
    )(gsx, gsr, gsw, gex, ger, gew, wrT, wwT, bc, wtsT, bts, wteT, bte,
      eid[:, None], disc.reshape(1, NEXER))

    xk = x[NSTU + NEXER:]
    ospec = pl.BlockSpec((NLAYER + 1, 2 * NKNOW, D),
                         lambda i: (0, NNODE // (2 * NKNOW) - 1, 0))
    knowledge_ts = pl.pallas_call(
        _tc_know_body,
        grid=(1,),
        in_specs=[ospec, ospec,
                  pl.BlockSpec((NKNOW, D), lambda i: (0, 0)),
                  pl.BlockSpec((D, D), lambda i: (0, 0)),
                  pl.BlockSpec((D, D), lambda i: (0, 0)),
                  pl.BlockSpec((1, D), lambda i: (0, 0)),
                  pl.BlockSpec((D, NKNOW), lambda i: (0, 0)),
                  pl.BlockSpec((1, NKNOW), lambda i: (0, 0))],
        out_specs=pl.BlockSpec((NKNOW, NKNOW), lambda i: (0, 0)),
        out_shape=jax.ShapeDtypeStruct((NKNOW, NKNOW), jnp.float32),
    )(o_r, o_w, xk, wrT, wwT, bc, wtkT, btk)

    return (student_ts, diff_ts, disc_ts, knowledge_ts)
